# Initial kernel scaffold; baseline (speedup 1.0000x reference)
import jax
import jax.numpy as jnp
from jax.experimental import pallas as pl


def kernel(x, edge_index, edge_weight, W1, b1, W2, b2, g1, be1, g2, be2):
    raise NotImplementedError("write your pallas kernel here")


# speedup vs baseline: 6.6945x; 6.6945x over previous
import functools

import jax
import jax.numpy as jnp
from jax import lax
from jax.experimental import pallas as pl
from jax.experimental.pallas import tpu as pltpu
from jax.experimental.pallas import tpu_sc as plsc

NC = 2
NS = 16
L = 16
NW = NC * NS


def _make_sc_scatter(N, D, E, C):
    EPW = E // NW
    NCH = EPW // C
    mesh = plsc.VectorSubcoreMesh(
        core_axis_name="c", subcore_axis_name="s",
        num_cores=NC, num_subcores=NS)
    RPT = (N // NS) // 8 * 8
    TAIL = N - NS * RPT

    @functools.partial(
        pl.kernel,
        out_type=jax.ShapeDtypeStruct((NC, N, D), jnp.float32),
        mesh=mesh,
        scratch_types=[
            pltpu.VMEM((EPW,), jnp.int32),
            pltpu.VMEM((NCH, C), jnp.int32),
            pltpu.VMEM((EPW,), jnp.float32),
            pltpu.VMEM((C, D), jnp.float32),
            pltpu.VMEM_SHARED((N, D), jnp.float32),
            pltpu.SemaphoreType.DMA,
        ],
    )
    def sc_scatter(x_hbm, src_hbm, dst_hbm, w_hbm, zeros_hbm, out_hbm,
                   src_v, dst_v, w_v, rows_v, agg_sh, sem):
        cid = lax.axis_index("c")
        sid = lax.axis_index("s")
        wid = sid * NC + cid
        base = wid * EPW
        pltpu.sync_copy(src_hbm.at[pl.ds(base, EPW)], src_v)
        pltpu.sync_copy(dst_hbm.at[wid], dst_v)
        pltpu.sync_copy(w_hbm.at[pl.ds(base, EPW)], w_v)
        pltpu.sync_copy(zeros_hbm.at[pl.ds(sid * RPT, RPT)],
                        agg_sh.at[pl.ds(sid * RPT, RPT)])
        if TAIL:
            @pl.when(sid == NS - 1)
            def _zero_tail():
                pltpu.sync_copy(zeros_hbm.at[pl.ds(NS * RPT, TAIL)],
                                agg_sh.at[pl.ds(NS * RPT, TAIL)])
        plsc.subcore_barrier()

        def chunk(j, carry):
            pltpu.async_copy(x_hbm.at[src_v.at[pl.ds(j * C, C)]],
                             rows_v, sem).wait()

            def group(g, carry2):
                w16 = w_v[pl.ds(j * C + g * L, L)]
                for l in range(L):
                    wspl = jnp.broadcast_to(w16[l], (L,))
                    e = g * L + l
                    for cg in range(D // L):
                        sl = (e, pl.ds(cg * L, L))
                        rows_v[sl] = rows_v[sl] * wspl
                return carry2

            lax.fori_loop(0, C // L, group, 0)
            pltpu.sync_copy(rows_v, agg_sh.at[dst_v.at[j]], add=True)
            return carry

        lax.fori_loop(0, NCH, chunk, 0)
        plsc.subcore_barrier()
        pltpu.sync_copy(agg_sh.at[pl.ds(sid * RPT, RPT)],
                        out_hbm.at[cid, pl.ds(sid * RPT, RPT)])
        if TAIL:
            @pl.when(sid == NS - 1)
            def _write_tail():
                pltpu.sync_copy(agg_sh.at[pl.ds(NS * RPT, TAIL)],
                                out_hbm.at[cid, pl.ds(NS * RPT, TAIL)])

    return sc_scatter


def _mlp_body(agg2_ref, w1_ref, b1_ref, w2_ref, b2_ref,
              g1_ref, be1_ref, g2_ref, be2_ref, out_ref):
    agg = agg2_ref[0] + agg2_ref[1]
    h = jnp.dot(agg, w1_ref[...], preferred_element_type=jnp.float32)
    h = h + b1_ref[...]
    mu = jnp.mean(h, axis=0, keepdims=True)
    var = jnp.mean((h - mu) * (h - mu), axis=0, keepdims=True)
    h = g1_ref[...] * (h - mu) * lax.rsqrt(var + 1e-5) + be1_ref[...]
    h = jnp.maximum(h, 0.0)
    h = jnp.dot(h, w2_ref[...], preferred_element_type=jnp.float32)
    h = h + b2_ref[...]
    mu2 = jnp.mean(h, axis=0, keepdims=True)
    var2 = jnp.mean((h - mu2) * (h - mu2), axis=0, keepdims=True)
    h = g2_ref[...] * (h - mu2) * lax.rsqrt(var2 + 1e-5) + be2_ref[...]
    out_ref[...] = jnp.maximum(h, 0.0)


def kernel(x, edge_index, edge_weight, W1, b1, W2, b2, g1, be1, g2, be2):
    N, D = x.shape
    E = edge_index.shape[1]
    C = 80
    src = edge_index[0]
    dst = edge_index[1].reshape(NW, (E // NW) // C, C)
    zeros = jnp.zeros((N, D), jnp.float32)

    agg2 = _make_sc_scatter(N, D, E, C)(x, src, dst, edge_weight, zeros)

    mlp = pl.pallas_call(
        _mlp_body,
        out_shape=jax.ShapeDtypeStruct((N, D), jnp.float32),
    )
    return mlp(agg2, W1, b1.reshape(1, D), W2, b2.reshape(1, D),
               g1.reshape(1, D), be1.reshape(1, D),
               g2.reshape(1, D), be2.reshape(1, D))
